# baseline (speedup 1.0000x reference)
import functools

import jax
import jax.numpy as jnp
from jax import lax
from jax.experimental import pallas as pl
from jax.experimental.pallas import tpu as pltpu
from jax.experimental.pallas import tpu_sc as plsc

N = 10000
E = 320000
D = 128
H = 128

NC = 2
NS = 16
NW = NC * NS

NPAD = 10240
RPT = NPAD // NS
EC = 128
NCH = 80
EW = NCH * EC
EPAD = NW * EW

RB = NPAD // NW
CB = 160

@functools.partial(
    pl.kernel,
    out_type=[
        jax.ShapeDtypeStruct((NC, NPAD, D), jnp.float32),
        jax.ShapeDtypeStruct((NC * NPAD,), jnp.float32),
    ],
    mesh=plsc.VectorSubcoreMesh(
        core_axis_name="c", subcore_axis_name="s", num_cores=NC,
        num_subcores=NS),
    scratch_types=[
        pltpu.VMEM((EW,), jnp.int32),
        pltpu.VMEM((EC,), jnp.int32),
        pltpu.VMEM((EC,), jnp.int32),
        pltpu.VMEM((EC, D), jnp.float32),
        pltpu.VMEM((EC, D), jnp.float32),
        pltpu.VMEM((EC,), jnp.float32),
        pltpu.VMEM((RPT,), jnp.float32),
        pltpu.VMEM_SHARED((NPAD, D), jnp.float32),
        pltpu.VMEM_SHARED((NPAD,), jnp.float32),
        pltpu.SemaphoreType.DMA,
        pltpu.SemaphoreType.DMA,
        pltpu.SemaphoreType.DMA,
        pltpu.SemaphoreType.DMA,
        pltpu.SemaphoreType.DMA,
        pltpu.SemaphoreType.DMA,
        pltpu.SemaphoreType.DMA,
    ],
)
def _sc_accumulate(x_hbm, src_hbm, dst_hbm, psum_hbm, pdeg_hbm,
                   sbuf, didx0, didx1, rows0, rows1, ones, dzero, acc, dacc,
                   gsem0, gsem1, ssem0, ssem1, isem0, isem1, dsem):
  cid = lax.axis_index("c")
  sid = lax.axis_index("s")
  wid = cid * NS + sid
  base = wid * EW

  pltpu.sync_copy(src_hbm.at[pl.ds(base, EW)], sbuf)

  def _zrow(r, _):
    for c in range(D // 16):
      rows0[r, pl.ds(c * 16, 16)] = jnp.zeros((16,), jnp.float32)
    return _
  lax.fori_loop(0, EC, _zrow, None)
  for k in range(RPT // EC):
    pltpu.sync_copy(rows0, acc.at[pl.ds(sid * RPT + k * EC, EC)])

  def _zdeg(i, _):
    dzero[pl.ds(i * 16, 16)] = jnp.zeros((16,), jnp.float32)
    return _
  lax.fori_loop(0, RPT // 16, _zdeg, None)
  pltpu.sync_copy(dzero, dacc.at[pl.ds(sid * RPT, RPT)])

  for i in range(EC // 16):
    ones[pl.ds(i * 16, 16)] = jnp.ones((16,), jnp.float32)

  plsc.subcore_barrier()

  bufs = ((rows0, didx0, gsem0, ssem0, isem0),
          (rows1, didx1, gsem1, ssem1, isem1))

  pltpu.sync_copy(dst_hbm.at[pl.ds(base, EC)], didx0)
  pltpu.async_copy(x_hbm.at[sbuf.at[pl.ds(0, EC)]], rows0, gsem0)

  def _step(i, b):
    rows, didx, gsem, ssem, isem = bufs[b]
    other_rows, other_didx, other_gsem, other_ssem, other_isem = bufs[1 - b]
    pltpu.make_async_copy(x_hbm.at[sbuf.at[pl.ds(i * EC, EC)]], rows,
                          gsem).wait()

    @pl.when(i > 0)
    def _():
      pltpu.make_async_copy(dst_hbm.at[pl.ds(base, EC)], didx, isem).wait()

    pltpu.async_copy(rows, acc.at[didx], ssem, add=True)
    pltpu.async_copy(ones, dacc.at[didx], dsem, add=True)

    @pl.when(i > 0)
    def _():
      pltpu.make_async_copy(other_rows, acc.at[didx], other_ssem).wait()
      pltpu.make_async_copy(ones, dacc.at[didx], dsem).wait()

    @pl.when(i + 1 < NCH)
    def _():
      pltpu.async_copy(x_hbm.at[sbuf.at[pl.ds((i + 1) * EC, EC)]],
                       other_rows, other_gsem)
      pltpu.async_copy(dst_hbm.at[pl.ds(base + (i + 1) * EC, EC)],
                       other_didx, other_isem)

  def _pair(g, _):
    _step(2 * g, 0)
    _step(2 * g + 1, 1)
    return _
  lax.fori_loop(0, NCH // 2, _pair, None)

  pltpu.make_async_copy(rows1, acc.at[didx1], ssem1).wait()
  pltpu.make_async_copy(ones, dacc.at[didx1], dsem).wait()

  plsc.subcore_barrier()

  sl = pl.ds(sid * RPT, RPT)
  pltpu.sync_copy(acc.at[sl], psum_hbm.at[cid, sl])
  pltpu.sync_copy(dacc.at[sl], pdeg_hbm.at[pl.ds(cid * NPAD + sid * RPT, RPT)])


@functools.partial(
    pl.kernel,
    out_type=jax.ShapeDtypeStruct((NPAD, D), jnp.float32),
    mesh=plsc.VectorSubcoreMesh(
        core_axis_name="c", subcore_axis_name="s", num_cores=NC,
        num_subcores=NS),
    scratch_types=[
        pltpu.VMEM((CB, D), jnp.float32),
        pltpu.VMEM((CB, D), jnp.float32),
        pltpu.VMEM((CB, D), jnp.float32),
        pltpu.VMEM((CB, D), jnp.float32),
        pltpu.VMEM((RB,), jnp.float32),
        pltpu.VMEM((RB,), jnp.float32),
    ],
)
def _sc_combine(x_hbm, psum_hbm, pdeg_hbm, agg_hbm,
                xb, p0b, p1b, ob, d0b, d1b):
  cid = lax.axis_index("c")
  sid = lax.axis_index("s")
  wid = cid * NS + sid
  base = wid * RB

  pltpu.sync_copy(pdeg_hbm.at[pl.ds(base, RB)], d0b)
  pltpu.sync_copy(pdeg_hbm.at[pl.ds(NPAD + base, RB)], d1b)

  for ch in range(RB // CB):
    rb = base + ch * CB
    pltpu.sync_copy(x_hbm.at[pl.ds(rb, CB)], xb)
    pltpu.sync_copy(psum_hbm.at[0, pl.ds(rb, CB)], p0b)
    pltpu.sync_copy(psum_hbm.at[1, pl.ds(rb, CB)], p1b)

    def _rowgrp(g, _):
      d = d0b[pl.ds(ch * CB + g * 16, 16)] + d1b[pl.ds(ch * CB + g * 16, 16)]
      has = d > 0.0
      sn = jnp.where(has, 0.5 / jnp.maximum(d, 1.0), 0.0)
      sx = jnp.where(has, 0.5, 1.0)
      for j in range(16):
        snj = sn[j]
        sxj = sx[j]
        for c in range(D // 16):
          sl = pl.ds(c * 16, 16)
          r = g * 16 + j
          ob[r, sl] = xb[r, sl] * sxj + (p0b[r, sl] + p1b[r, sl]) * snj
      return _
    lax.fori_loop(0, CB // 16, _rowgrp, None)

    pltpu.sync_copy(ob, agg_hbm.at[pl.ds(rb, CB)])


BN = 2000


def _mlp_body(a_ref, w1_ref, b1_ref, w2_ref, b2_ref, w3_ref, b3_ref, o_ref):
  dn = (((1,), (1,)), ((), ()))
  a = a_ref[...]
  h = lax.dot_general(a, w1_ref[...], dn, preferred_element_type=jnp.float32)
  h = jnp.maximum(h + b1_ref[...], 0.0)
  h = lax.dot_general(h, w2_ref[...], dn, preferred_element_type=jnp.float32)
  h = jnp.maximum(h + b2_ref[...], 0.0)
  o_ref[...] = jnp.sum(h * w3_ref[...], axis=1, keepdims=True) + b3_ref[...]


def _tc_mlp(agg, W1, b1, W2, b2, W3, b3):
  return pl.pallas_call(
      _mlp_body,
      grid=(N // BN,),
      in_specs=[
          pl.BlockSpec((BN, D), lambda g: (g, 0)),
          pl.BlockSpec((H, D), lambda g: (0, 0)),
          pl.BlockSpec((1, H), lambda g: (0, 0)),
          pl.BlockSpec((H, H), lambda g: (0, 0)),
          pl.BlockSpec((1, H), lambda g: (0, 0)),
          pl.BlockSpec((1, H), lambda g: (0, 0)),
          pl.BlockSpec((1, 1), lambda g: (0, 0)),
      ],
      out_specs=pl.BlockSpec((BN, 1), lambda g: (g, 0)),
      out_shape=jax.ShapeDtypeStruct((N, 1), jnp.float32),
  )(agg, W1, b1.reshape(1, H), W2, b2.reshape(1, H), W3, b3.reshape(1, 1))


def kernel(x, edge_index, W1, b1, W2, b2, W3, b3):
  src = jnp.pad(edge_index[0], (0, EPAD - E))
  dst = jnp.pad(edge_index[1], (0, EPAD - E), constant_values=NPAD - 1)
  x_pad = jnp.pad(x, ((0, NPAD - N), (0, 0)))
  psum, pdeg = _sc_accumulate(x_pad, src, dst)
  agg = _sc_combine(x_pad, psum, pdeg)
  return _tc_mlp(agg[:N], W1, b1, W2, b2, W3, b3)


# speedup vs baseline: 1.0012x; 1.0012x over previous
import functools

import jax
import jax.numpy as jnp
from jax import lax
from jax.experimental import pallas as pl
from jax.experimental.pallas import tpu as pltpu
from jax.experimental.pallas import tpu_sc as plsc

N = 10000
E = 320000
D = 128
H = 128

NC = 2
NS = 16
NW = NC * NS

NPAD = 10240
RPT = NPAD // NS
EC = 128
NCH = 80
EW = NCH * EC
EPAD = NW * EW

RB = NPAD // NW
CB = 160

@functools.partial(
    pl.kernel,
    out_type=[
        jax.ShapeDtypeStruct((NC, NPAD, D), jnp.float32),
        jax.ShapeDtypeStruct((NC * NPAD,), jnp.float32),
    ],
    mesh=plsc.VectorSubcoreMesh(
        core_axis_name="c", subcore_axis_name="s", num_cores=NC,
        num_subcores=NS),
    scratch_types=[
        pltpu.VMEM((EW,), jnp.int32),
        pltpu.VMEM((EC,), jnp.int32),
        pltpu.VMEM((EC,), jnp.int32),
        pltpu.VMEM((EC, D), jnp.float32),
        pltpu.VMEM((EC, D), jnp.float32),
        pltpu.VMEM((EC,), jnp.float32),
        pltpu.VMEM((RPT,), jnp.float32),
        pltpu.VMEM_SHARED((NPAD, D), jnp.float32),
        pltpu.VMEM_SHARED((NPAD,), jnp.float32),
        pltpu.SemaphoreType.DMA,
        pltpu.SemaphoreType.DMA,
        pltpu.SemaphoreType.DMA,
        pltpu.SemaphoreType.DMA,
        pltpu.SemaphoreType.DMA,
        pltpu.SemaphoreType.DMA,
        pltpu.SemaphoreType.DMA,
    ],
)
def _sc_accumulate(x_hbm, src_hbm, dst_hbm, psum_hbm, pdeg_hbm,
                   sbuf, didx0, didx1, rows0, rows1, ones, dzero, acc, dacc,
                   gsem0, gsem1, ssem0, ssem1, isem0, isem1, dsem):
  cid = lax.axis_index("c")
  sid = lax.axis_index("s")
  wid = cid * NS + sid
  base = wid * EW

  pltpu.sync_copy(src_hbm.at[pl.ds(base, EW)], sbuf)

  def _zrow(r, _):
    for c in range(D // 16):
      rows0[r, pl.ds(c * 16, 16)] = jnp.zeros((16,), jnp.float32)
    return _
  lax.fori_loop(0, EC, _zrow, None)
  for k in range(RPT // EC):
    pltpu.sync_copy(rows0, acc.at[pl.ds(sid * RPT + k * EC, EC)])

  def _zdeg(i, _):
    dzero[pl.ds(i * 16, 16)] = jnp.zeros((16,), jnp.float32)
    return _
  lax.fori_loop(0, RPT // 16, _zdeg, None)
  pltpu.sync_copy(dzero, dacc.at[pl.ds(sid * RPT, RPT)])

  for i in range(EC // 16):
    ones[pl.ds(i * 16, 16)] = jnp.ones((16,), jnp.float32)

  plsc.subcore_barrier()

  bufs = ((rows0, didx0, gsem0, ssem0, isem0),
          (rows1, didx1, gsem1, ssem1, isem1))

  pltpu.sync_copy(dst_hbm.at[pl.ds(base, EC)], didx0)
  pltpu.async_copy(x_hbm.at[sbuf.at[pl.ds(0, EC)]], rows0, gsem0)

  def _step(i, b):
    rows, didx, gsem, ssem, isem = bufs[b]
    other_rows, other_didx, other_gsem, other_ssem, other_isem = bufs[1 - b]
    pltpu.make_async_copy(x_hbm.at[sbuf.at[pl.ds(i * EC, EC)]], rows,
                          gsem).wait()

    @pl.when(i > 0)
    def _():
      pltpu.make_async_copy(dst_hbm.at[pl.ds(base, EC)], didx, isem).wait()

    pltpu.async_copy(rows, acc.at[didx], ssem, add=True)
    pltpu.async_copy(ones, dacc.at[didx], dsem, add=True)

    @pl.when(i > 0)
    def _():
      pltpu.make_async_copy(other_rows, acc.at[didx], other_ssem).wait()
      pltpu.make_async_copy(ones, dacc.at[didx], dsem).wait()

    @pl.when(i + 1 < NCH)
    def _():
      pltpu.async_copy(x_hbm.at[sbuf.at[pl.ds((i + 1) * EC, EC)]],
                       other_rows, other_gsem)
      pltpu.async_copy(dst_hbm.at[pl.ds(base + (i + 1) * EC, EC)],
                       other_didx, other_isem)

  def _pair(g, _):
    _step(2 * g, 0)
    _step(2 * g + 1, 1)
    return _
  lax.fori_loop(0, NCH // 2, _pair, None)

  pltpu.make_async_copy(rows1, acc.at[didx1], ssem1).wait()
  pltpu.make_async_copy(ones, dacc.at[didx1], dsem).wait()

  plsc.subcore_barrier()

  sl = pl.ds(sid * RPT, RPT)
  pltpu.sync_copy(acc.at[sl], psum_hbm.at[cid, sl])
  pltpu.sync_copy(dacc.at[sl], pdeg_hbm.at[pl.ds(cid * NPAD + sid * RPT, RPT)])


@functools.partial(
    pl.kernel,
    out_type=jax.ShapeDtypeStruct((NPAD, D), jnp.float32),
    mesh=plsc.VectorSubcoreMesh(
        core_axis_name="c", subcore_axis_name="s", num_cores=NC,
        num_subcores=NS),
    scratch_types=[
        pltpu.VMEM((CB, D), jnp.float32),
        pltpu.VMEM((CB, D), jnp.float32),
        pltpu.VMEM((CB, D), jnp.float32),
        pltpu.VMEM((CB, D), jnp.float32),
        pltpu.VMEM((RB,), jnp.float32),
        pltpu.VMEM((RB,), jnp.float32),
    ],
)
def _sc_combine(x_hbm, psum_hbm, pdeg_hbm, agg_hbm,
                xb, p0b, p1b, ob, d0b, d1b):
  cid = lax.axis_index("c")
  sid = lax.axis_index("s")
  wid = cid * NS + sid
  base = wid * RB

  pltpu.sync_copy(pdeg_hbm.at[pl.ds(base, RB)], d0b)
  pltpu.sync_copy(pdeg_hbm.at[pl.ds(NPAD + base, RB)], d1b)

  for ch in range(RB // CB):
    rb = base + ch * CB
    pltpu.sync_copy(x_hbm.at[pl.ds(rb, CB)], xb)
    pltpu.sync_copy(psum_hbm.at[0, pl.ds(rb, CB)], p0b)
    pltpu.sync_copy(psum_hbm.at[1, pl.ds(rb, CB)], p1b)

    def _rowgrp(g, _):
      d = d0b[pl.ds(ch * CB + g * 16, 16)] + d1b[pl.ds(ch * CB + g * 16, 16)]
      has = d > 0.0
      sn = jnp.where(has, 0.5 / jnp.maximum(d, 1.0), 0.0)
      sx = jnp.where(has, 0.5, 1.0)
      for j in range(16):
        snj = sn[j]
        sxj = sx[j]
        for c in range(D // 16):
          sl = pl.ds(c * 16, 16)
          r = g * 16 + j
          ob[r, sl] = xb[r, sl] * sxj + (p0b[r, sl] + p1b[r, sl]) * snj
      return _
    lax.fori_loop(0, CB // 16, _rowgrp, None)

    pltpu.sync_copy(ob, agg_hbm.at[pl.ds(rb, CB)])


BN = 2000


def _mlp_body(a_ref, w1_ref, b1_ref, w2_ref, b2_ref, w3_ref, b3_ref, o_ref):
  dn = (((1,), (1,)), ((), ()))
  a = a_ref[...]
  h = lax.dot_general(a, w1_ref[...], dn, preferred_element_type=jnp.float32)
  h = jnp.maximum(h + b1_ref[...], 0.0)
  h = lax.dot_general(h, w2_ref[...], dn, preferred_element_type=jnp.float32)
  h = jnp.maximum(h + b2_ref[...], 0.0)
  o_ref[...] = jnp.sum(h * w3_ref[...], axis=1, keepdims=True) + b3_ref[...]


def _tc_mlp(agg, W1, b1, W2, b2, W3, b3):
  return pl.pallas_call(
      _mlp_body,
      grid=(N // BN,),
      in_specs=[
          pl.BlockSpec((BN, D), lambda g: (g, 0)),
          pl.BlockSpec((H, D), lambda g: (0, 0)),
          pl.BlockSpec((1, H), lambda g: (0, 0)),
          pl.BlockSpec((H, H), lambda g: (0, 0)),
          pl.BlockSpec((1, H), lambda g: (0, 0)),
          pl.BlockSpec((1, H), lambda g: (0, 0)),
          pl.BlockSpec((1, 1), lambda g: (0, 0)),
      ],
      out_specs=pl.BlockSpec((BN, 1), lambda g: (g, 0)),
      out_shape=jax.ShapeDtypeStruct((N, 1), jnp.float32),
  )(agg, W1, b1.reshape(1, H), W2, b2.reshape(1, H), W3, b3.reshape(1, 1))


def kernel(x, edge_index, W1, b1, W2, b2, W3, b3):
  src = jnp.pad(edge_index[0], (0, EPAD - E))
  pad_dst = N + (jnp.arange(EPAD - E, dtype=jnp.int32) % (NPAD - N))
  dst = jnp.concatenate([edge_index[1], pad_dst])
  x_pad = jnp.pad(x, ((0, NPAD - N), (0, 0)))
  psum, pdeg = _sc_accumulate(x_pad, src, dst)
  agg = _sc_combine(x_pad, psum, pdeg)
  return _tc_mlp(agg[:N], W1, b1, W2, b2, W3, b3)


# speedup vs baseline: 1.9493x; 1.9470x over previous
import functools

import jax
import jax.numpy as jnp
from jax import lax
from jax.experimental import pallas as pl
from jax.experimental.pallas import tpu as pltpu
from jax.experimental.pallas import tpu_sc as plsc

N = 10000
E = 320000
D = 128
H = 128

NC = 2
NS = 16
NW = NC * NS

NPAD = 10240
RPT = NPAD // NS
EW = E // NW
EC = 80
NCH = EW // EC

RB = NPAD // NW
CB = 160

@functools.partial(
    pl.kernel,
    out_type=[
        jax.ShapeDtypeStruct((NC, NPAD, D), jnp.float32),
        jax.ShapeDtypeStruct((NC * NPAD,), jnp.float32),
    ],
    mesh=plsc.VectorSubcoreMesh(
        core_axis_name="c", subcore_axis_name="s", num_cores=NC,
        num_subcores=NS),
    scratch_types=[
        pltpu.VMEM((EW,), jnp.int32),
        pltpu.VMEM((NCH, EC), jnp.int32),
        pltpu.VMEM((EC, D), jnp.float32),
        pltpu.VMEM((EC, D), jnp.float32),
        pltpu.VMEM((EC,), jnp.float32),
        pltpu.VMEM((RPT,), jnp.float32),
        pltpu.VMEM_SHARED((NPAD, D), jnp.float32),
        pltpu.VMEM_SHARED((NPAD,), jnp.float32),
        pltpu.SemaphoreType.DMA,
        pltpu.SemaphoreType.DMA,
        pltpu.SemaphoreType.DMA,
        pltpu.SemaphoreType.DMA,
        pltpu.SemaphoreType.DMA,
    ],
)
def _sc_accumulate(x_hbm, src_hbm, dst_hbm, psum_hbm, pdeg_hbm,
                   sbuf, dbuf, rows0, rows1, ones, dzero, acc, dacc,
                   gsem0, gsem1, ssem0, ssem1, dsem):
  cid = lax.axis_index("c")
  sid = lax.axis_index("s")
  wid = cid * NS + sid

  pltpu.sync_copy(src_hbm.at[pl.ds(wid * EW, EW)], sbuf)
  pltpu.sync_copy(dst_hbm.at[wid], dbuf)

  def _zrow(r, _):
    for c in range(D // 16):
      rows0[r, pl.ds(c * 16, 16)] = jnp.zeros((16,), jnp.float32)
    return _
  lax.fori_loop(0, EC, _zrow, None)
  for k in range(RPT // EC):
    pltpu.sync_copy(rows0, acc.at[pl.ds(sid * RPT + k * EC, EC)])

  def _zdeg(i, _):
    dzero[pl.ds(i * 16, 16)] = jnp.zeros((16,), jnp.float32)
    return _
  lax.fori_loop(0, RPT // 16, _zdeg, None)
  pltpu.sync_copy(dzero, dacc.at[pl.ds(sid * RPT, RPT)])

  for i in range(EC // 16):
    ones[pl.ds(i * 16, 16)] = jnp.ones((16,), jnp.float32)

  plsc.subcore_barrier()

  bufs = ((rows0, gsem0, ssem0), (rows1, gsem1, ssem1))

  pltpu.async_copy(x_hbm.at[sbuf.at[pl.ds(0, EC)]], rows0, gsem0)

  def _step(i, b):
    rows, gsem, ssem = bufs[b]
    pltpu.make_async_copy(x_hbm.at[sbuf.at[pl.ds(i * EC, EC)]], rows,
                          gsem).wait()
    pltpu.async_copy(rows, acc.at[dbuf.at[i]], ssem, add=True)
    pltpu.async_copy(ones, dacc.at[dbuf.at[i]], dsem, add=True)

    other_rows, other_gsem, other_ssem = bufs[1 - b]

    @pl.when(i > 0)
    def _():
      pltpu.make_async_copy(other_rows, acc.at[dbuf.at[i]], other_ssem).wait()
      pltpu.make_async_copy(ones, dacc.at[dbuf.at[i]], dsem).wait()

    @pl.when(i + 1 < NCH)
    def _():
      pltpu.async_copy(x_hbm.at[sbuf.at[pl.ds((i + 1) * EC, EC)]],
                       other_rows, other_gsem)

  def _pair(g, _):
    _step(2 * g, 0)
    _step(2 * g + 1, 1)
    return _
  lax.fori_loop(0, NCH // 2, _pair, None)
  _step(NCH - 1, 0)

  pltpu.make_async_copy(rows0, acc.at[dbuf.at[0]], ssem0).wait()
  pltpu.make_async_copy(ones, dacc.at[dbuf.at[0]], dsem).wait()

  plsc.subcore_barrier()

  sl = pl.ds(sid * RPT, RPT)
  pltpu.sync_copy(acc.at[sl], psum_hbm.at[cid, sl])
  pltpu.sync_copy(dacc.at[sl], pdeg_hbm.at[pl.ds(cid * NPAD + sid * RPT, RPT)])


@functools.partial(
    pl.kernel,
    out_type=jax.ShapeDtypeStruct((NPAD, D), jnp.float32),
    mesh=plsc.VectorSubcoreMesh(
        core_axis_name="c", subcore_axis_name="s", num_cores=NC,
        num_subcores=NS),
    scratch_types=[
        pltpu.VMEM((CB, D), jnp.float32),
        pltpu.VMEM((CB, D), jnp.float32),
        pltpu.VMEM((CB, D), jnp.float32),
        pltpu.VMEM((CB, D), jnp.float32),
        pltpu.VMEM((RB,), jnp.float32),
        pltpu.VMEM((RB,), jnp.float32),
    ],
)
def _sc_combine(x_hbm, psum_hbm, pdeg_hbm, agg_hbm,
                xb, p0b, p1b, ob, d0b, d1b):
  cid = lax.axis_index("c")
  sid = lax.axis_index("s")
  wid = cid * NS + sid
  base = wid * RB

  pltpu.sync_copy(pdeg_hbm.at[pl.ds(base, RB)], d0b)
  pltpu.sync_copy(pdeg_hbm.at[pl.ds(NPAD + base, RB)], d1b)

  for ch in range(RB // CB):
    rb = base + ch * CB
    pltpu.sync_copy(x_hbm.at[pl.ds(rb, CB)], xb)
    pltpu.sync_copy(psum_hbm.at[0, pl.ds(rb, CB)], p0b)
    pltpu.sync_copy(psum_hbm.at[1, pl.ds(rb, CB)], p1b)

    def _rowgrp(g, _):
      d = d0b[pl.ds(ch * CB + g * 16, 16)] + d1b[pl.ds(ch * CB + g * 16, 16)]
      has = d > 0.0
      sn = jnp.where(has, 0.5 / jnp.maximum(d, 1.0), 0.0)
      sx = jnp.where(has, 0.5, 1.0)
      for j in range(16):
        snj = sn[j]
        sxj = sx[j]
        for c in range(D // 16):
          sl = pl.ds(c * 16, 16)
          r = g * 16 + j
          ob[r, sl] = xb[r, sl] * sxj + (p0b[r, sl] + p1b[r, sl]) * snj
      return _
    lax.fori_loop(0, CB // 16, _rowgrp, None)

    pltpu.sync_copy(ob, agg_hbm.at[pl.ds(rb, CB)])


BN = 2000


def _mlp_body(a_ref, w1_ref, b1_ref, w2_ref, b2_ref, w3_ref, b3_ref, o_ref):
  dn = (((1,), (1,)), ((), ()))
  a = a_ref[...]
  h = lax.dot_general(a, w1_ref[...], dn, preferred_element_type=jnp.float32)
  h = jnp.maximum(h + b1_ref[...], 0.0)
  h = lax.dot_general(h, w2_ref[...], dn, preferred_element_type=jnp.float32)
  h = jnp.maximum(h + b2_ref[...], 0.0)
  o_ref[...] = jnp.sum(h * w3_ref[...], axis=1, keepdims=True) + b3_ref[...]


def _tc_mlp(agg, W1, b1, W2, b2, W3, b3):
  return pl.pallas_call(
      _mlp_body,
      grid=(N // BN,),
      in_specs=[
          pl.BlockSpec((BN, D), lambda g: (g, 0)),
          pl.BlockSpec((H, D), lambda g: (0, 0)),
          pl.BlockSpec((1, H), lambda g: (0, 0)),
          pl.BlockSpec((H, H), lambda g: (0, 0)),
          pl.BlockSpec((1, H), lambda g: (0, 0)),
          pl.BlockSpec((1, H), lambda g: (0, 0)),
          pl.BlockSpec((1, 1), lambda g: (0, 0)),
      ],
      out_specs=pl.BlockSpec((BN, 1), lambda g: (g, 0)),
      out_shape=jax.ShapeDtypeStruct((N, 1), jnp.float32),
  )(agg, W1, b1.reshape(1, H), W2, b2.reshape(1, H), W3, b3.reshape(1, 1))


def kernel(x, edge_index, W1, b1, W2, b2, W3, b3):
  src = edge_index[0]
  dst = edge_index[1].reshape(NW, NCH, EC)
  x_pad = jnp.pad(x, ((0, NPAD - N), (0, 0)))
  psum, pdeg = _sc_accumulate(x_pad, src, dst)
  agg = _sc_combine(x_pad, psum, pdeg)
  return _tc_mlp(agg[:N], W1, b1, W2, b2, W3, b3)


# speedup vs baseline: 2.5269x; 1.2963x over previous
import functools

import jax
import jax.numpy as jnp
from jax import lax
from jax.experimental import pallas as pl
from jax.experimental.pallas import tpu as pltpu
from jax.experimental.pallas import tpu_sc as plsc

N = 10000
E = 320000
D = 128
H = 128

NC = 2
NS = 16
NW = NC * NS

NPAD = 10240
RPT = NPAD // NS
EW = E // NW
EC = 80
NCH = EW // EC

RB = NPAD // NW
CB = 160

@functools.partial(
    pl.kernel,
    out_type=[
        jax.ShapeDtypeStruct((NC, NPAD, D), jnp.float32),
        jax.ShapeDtypeStruct((NC * NPAD,), jnp.float32),
    ],
    mesh=plsc.VectorSubcoreMesh(
        core_axis_name="c", subcore_axis_name="s", num_cores=NC,
        num_subcores=NS),
    scratch_types=[
        pltpu.VMEM((NCH, EC), jnp.int32),
        pltpu.VMEM((EC,), jnp.int32),
        pltpu.VMEM((EC,), jnp.int32),
        pltpu.VMEM((EC,), jnp.int32),
        pltpu.VMEM((EC, D), jnp.float32),
        pltpu.VMEM((EC, D), jnp.float32),
        pltpu.VMEM((EC, D), jnp.float32),
        pltpu.VMEM((EC,), jnp.float32),
        pltpu.VMEM((RPT,), jnp.float32),
        pltpu.VMEM_SHARED((NPAD, D), jnp.float32),
        pltpu.VMEM_SHARED((NPAD,), jnp.float32),
        pltpu.SemaphoreType.DMA,
        pltpu.SemaphoreType.DMA,
        pltpu.SemaphoreType.DMA,
        pltpu.SemaphoreType.DMA,
        pltpu.SemaphoreType.DMA,
        pltpu.SemaphoreType.DMA,
        pltpu.SemaphoreType.DMA,
        pltpu.SemaphoreType.DMA,
        pltpu.SemaphoreType.DMA,
        pltpu.SemaphoreType.DMA,
    ],
)
def _sc_accumulate(x_hbm, src_hbm, dst_hbm, psum_hbm, pdeg_hbm,
                   dbuf, sidx0, sidx1, sidx2, rows0, rows1, rows2,
                   ones, dzero, acc, dacc,
                   gsem0, gsem1, gsem2, ssem0, ssem1, ssem2,
                   isem0, isem1, isem2, dsem):
  cid = lax.axis_index("c")
  sid = lax.axis_index("s")
  wid = cid * NS + sid
  base = wid * EW

  pltpu.sync_copy(dst_hbm.at[wid], dbuf)

  def _zrow(r, _):
    for c in range(D // 16):
      rows0[r, pl.ds(c * 16, 16)] = jnp.zeros((16,), jnp.float32)
    return _
  lax.fori_loop(0, EC, _zrow, None)
  for k in range(RPT // EC):
    pltpu.sync_copy(rows0, acc.at[pl.ds(sid * RPT + k * EC, EC)])

  def _zdeg(i, _):
    dzero[pl.ds(i * 16, 16)] = jnp.zeros((16,), jnp.float32)
    return _
  lax.fori_loop(0, RPT // 16, _zdeg, None)
  pltpu.sync_copy(dzero, dacc.at[pl.ds(sid * RPT, RPT)])

  for i in range(EC // 16):
    ones[pl.ds(i * 16, 16)] = jnp.ones((16,), jnp.float32)

  plsc.subcore_barrier()

  bufs = ((rows0, sidx0, gsem0, ssem0, isem0),
          (rows1, sidx1, gsem1, ssem1, isem1),
          (rows2, sidx2, gsem2, ssem2, isem2))

  pltpu.sync_copy(src_hbm.at[pl.ds(base, EC)], sidx0)
  pltpu.sync_copy(src_hbm.at[pl.ds(base + EC, EC)], sidx1)
  pltpu.async_copy(x_hbm.at[sidx0], rows0, gsem0)
  pltpu.async_copy(x_hbm.at[sidx1], rows1, gsem1)
  pltpu.async_copy(src_hbm.at[pl.ds(base + 2 * EC, EC)], sidx2, isem2)

  def _step(i, b):
    rows, sidx, gsem, ssem, isem = bufs[b]
    rows_p, sidx_p, gsem_p, ssem_p, isem_p = bufs[(b + 2) % 3]
    pltpu.make_async_copy(x_hbm.at[sidx], rows, gsem).wait()
    pltpu.async_copy(rows, acc.at[dbuf.at[i]], ssem, add=True)
    pltpu.async_copy(ones, dacc.at[dbuf.at[i]], dsem, add=True)

    @pl.when(i > 0)
    def _():
      pltpu.make_async_copy(rows_p, acc.at[dbuf.at[i]], ssem_p).wait()
      pltpu.make_async_copy(ones, dacc.at[dbuf.at[i]], dsem).wait()

    @pl.when(i + 2 < NCH)
    def _():
      pltpu.make_async_copy(src_hbm.at[pl.ds(base, EC)], sidx_p,
                            isem_p).wait()
      pltpu.async_copy(x_hbm.at[sidx_p], rows_p, gsem_p)

    @pl.when(i + 3 < NCH)
    def _():
      pltpu.async_copy(src_hbm.at[pl.ds(base + (i + 3) * EC, EC)],
                       sidx, isem)

  def _triple(g, _):
    _step(3 * g, 0)
    _step(3 * g + 1, 1)
    _step(3 * g + 2, 2)
    return _
  lax.fori_loop(0, NCH // 3, _triple, None)
  _step(NCH - 2, 0)
  _step(NCH - 1, 1)

  pltpu.make_async_copy(rows1, acc.at[dbuf.at[0]], ssem1).wait()
  pltpu.make_async_copy(ones, dacc.at[dbuf.at[0]], dsem).wait()

  plsc.subcore_barrier()

  sl = pl.ds(sid * RPT, RPT)
  pltpu.sync_copy(acc.at[sl], psum_hbm.at[cid, sl])
  pltpu.sync_copy(dacc.at[sl], pdeg_hbm.at[pl.ds(cid * NPAD + sid * RPT, RPT)])


@functools.partial(
    pl.kernel,
    out_type=jax.ShapeDtypeStruct((NPAD, D), jnp.float32),
    mesh=plsc.VectorSubcoreMesh(
        core_axis_name="c", subcore_axis_name="s", num_cores=NC,
        num_subcores=NS),
    scratch_types=[
        pltpu.VMEM((CB, D), jnp.float32),
        pltpu.VMEM((CB, D), jnp.float32),
        pltpu.VMEM((CB, D), jnp.float32),
        pltpu.VMEM((CB, D), jnp.float32),
        pltpu.VMEM((RB,), jnp.float32),
        pltpu.VMEM((RB,), jnp.float32),
    ],
)
def _sc_combine(x_hbm, psum_hbm, pdeg_hbm, agg_hbm,
                xb, p0b, p1b, ob, d0b, d1b):
  cid = lax.axis_index("c")
  sid = lax.axis_index("s")
  wid = cid * NS + sid
  base = wid * RB

  pltpu.sync_copy(pdeg_hbm.at[pl.ds(base, RB)], d0b)
  pltpu.sync_copy(pdeg_hbm.at[pl.ds(NPAD + base, RB)], d1b)

  for ch in range(RB // CB):
    rb = base + ch * CB
    pltpu.sync_copy(x_hbm.at[pl.ds(rb, CB)], xb)
    pltpu.sync_copy(psum_hbm.at[0, pl.ds(rb, CB)], p0b)
    pltpu.sync_copy(psum_hbm.at[1, pl.ds(rb, CB)], p1b)

    def _rowgrp(g, _):
      d = d0b[pl.ds(ch * CB + g * 16, 16)] + d1b[pl.ds(ch * CB + g * 16, 16)]
      has = d > 0.0
      sn = jnp.where(has, 0.5 / jnp.maximum(d, 1.0), 0.0)
      sx = jnp.where(has, 0.5, 1.0)
      for j in range(16):
        snj = sn[j]
        sxj = sx[j]
        for c in range(D // 16):
          sl = pl.ds(c * 16, 16)
          r = g * 16 + j
          ob[r, sl] = xb[r, sl] * sxj + (p0b[r, sl] + p1b[r, sl]) * snj
      return _
    lax.fori_loop(0, CB // 16, _rowgrp, None)

    pltpu.sync_copy(ob, agg_hbm.at[pl.ds(rb, CB)])


BN = 2000


def _mlp_body(a_ref, w1_ref, b1_ref, w2_ref, b2_ref, w3_ref, b3_ref, o_ref):
  dn = (((1,), (1,)), ((), ()))
  a = a_ref[...]
  h = lax.dot_general(a, w1_ref[...], dn, preferred_element_type=jnp.float32)
  h = jnp.maximum(h + b1_ref[...], 0.0)
  h = lax.dot_general(h, w2_ref[...], dn, preferred_element_type=jnp.float32)
  h = jnp.maximum(h + b2_ref[...], 0.0)
  o_ref[...] = jnp.sum(h * w3_ref[...], axis=1, keepdims=True) + b3_ref[...]


def _tc_mlp(agg, W1, b1, W2, b2, W3, b3):
  return pl.pallas_call(
      _mlp_body,
      grid=(N // BN,),
      in_specs=[
          pl.BlockSpec((BN, D), lambda g: (g, 0)),
          pl.BlockSpec((H, D), lambda g: (0, 0)),
          pl.BlockSpec((1, H), lambda g: (0, 0)),
          pl.BlockSpec((H, H), lambda g: (0, 0)),
          pl.BlockSpec((1, H), lambda g: (0, 0)),
          pl.BlockSpec((1, H), lambda g: (0, 0)),
          pl.BlockSpec((1, 1), lambda g: (0, 0)),
      ],
      out_specs=pl.BlockSpec((BN, 1), lambda g: (g, 0)),
      out_shape=jax.ShapeDtypeStruct((N, 1), jnp.float32),
  )(agg, W1, b1.reshape(1, H), W2, b2.reshape(1, H), W3, b3.reshape(1, 1))


def kernel(x, edge_index, W1, b1, W2, b2, W3, b3):
  src = edge_index[0]
  dst = edge_index[1].reshape(NW, NCH, EC)
  x_pad = jnp.pad(x, ((0, NPAD - N), (0, 0)))
  psum, pdeg = _sc_accumulate(x_pad, src, dst)
  agg = _sc_combine(x_pad, psum, pdeg)
  return _tc_mlp(agg[:N], W1, b1, W2, b2, W3, b3)


# speedup vs baseline: 2.6649x; 1.0546x over previous
import functools

import jax
import jax.numpy as jnp
from jax import lax
from jax.experimental import pallas as pl
from jax.experimental.pallas import tpu as pltpu
from jax.experimental.pallas import tpu_sc as plsc

N = 10000
E = 320000
D = 128
H = 128

NC = 2
NS = 16
NW = NC * NS

NPAD = 10240
RPT = NPAD // NS
EW = E // NW
EC = 80
NCH = EW // EC

RB = NPAD // NW
CB = 160

@functools.partial(
    pl.kernel,
    out_type=[
        jax.ShapeDtypeStruct((NC, NPAD, D), jnp.float32),
        jax.ShapeDtypeStruct((NC * NPAD,), jnp.float32),
    ],
    mesh=plsc.VectorSubcoreMesh(
        core_axis_name="c", subcore_axis_name="s", num_cores=NC,
        num_subcores=NS),
    scratch_types=[
        [pltpu.VMEM((EC,), jnp.int32)] * 4,
        [pltpu.VMEM((EC,), jnp.int32)] * 4,
        [pltpu.VMEM((EC, D), jnp.float32)] * 4,
        pltpu.VMEM((EC,), jnp.float32),
        pltpu.VMEM((RPT,), jnp.float32),
        pltpu.VMEM_SHARED((NPAD, D), jnp.float32),
        pltpu.VMEM_SHARED((NPAD,), jnp.float32),
        [pltpu.SemaphoreType.DMA] * 4,
        [pltpu.SemaphoreType.DMA] * 4,
        [pltpu.SemaphoreType.DMA] * 4,
        [pltpu.SemaphoreType.DMA] * 4,
        pltpu.SemaphoreType.DMA,
    ],
)
def _sc_accumulate(x_hbm, src_hbm, dst_hbm, psum_hbm, pdeg_hbm,
                   sidx, didx, rowsb, ones, dzero, acc, dacc,
                   gsem, ssem, isem, jsem, dsem):
  cid = lax.axis_index("c")
  sid = lax.axis_index("s")
  wid = cid * NS + sid
  base = wid * EW
  rows0 = rowsb[0]

  def _zrow(r, _):
    for c in range(D // 16):
      rows0[r, pl.ds(c * 16, 16)] = jnp.zeros((16,), jnp.float32)
    return _
  lax.fori_loop(0, EC, _zrow, None)
  for k in range(RPT // EC):
    pltpu.sync_copy(rows0, acc.at[pl.ds(sid * RPT + k * EC, EC)])

  def _zdeg(i, _):
    dzero[pl.ds(i * 16, 16)] = jnp.zeros((16,), jnp.float32)
    return _
  lax.fori_loop(0, RPT // 16, _zdeg, None)
  pltpu.sync_copy(dzero, dacc.at[pl.ds(sid * RPT, RPT)])

  for i in range(EC // 16):
    ones[pl.ds(i * 16, 16)] = jnp.ones((16,), jnp.float32)

  plsc.subcore_barrier()

  for k in range(3):
    pltpu.sync_copy(src_hbm.at[pl.ds(base + k * EC, EC)], sidx[k])
    pltpu.sync_copy(dst_hbm.at[pl.ds(base + k * EC, EC)], didx[k])
    pltpu.async_copy(x_hbm.at[sidx[k]], rowsb[k], gsem[k])
  pltpu.async_copy(src_hbm.at[pl.ds(base + 3 * EC, EC)], sidx[3], isem[3])

  def _step(i, b):
    p = (b + 3) % 4
    pltpu.make_async_copy(x_hbm.at[sidx[b]], rowsb[b], gsem[b]).wait()

    @pl.when(i >= 3)
    def _():
      pltpu.make_async_copy(dst_hbm.at[pl.ds(base, EC)], didx[b],
                            jsem[b]).wait()

    pltpu.async_copy(rowsb[b], acc.at[didx[b]], ssem[b], add=True)
    pltpu.async_copy(ones, dacc.at[didx[b]], dsem, add=True)

    @pl.when(i > 0)
    def _():
      pltpu.make_async_copy(rowsb[p], acc.at[didx[p]], ssem[p]).wait()
      pltpu.make_async_copy(ones, dacc.at[didx[p]], dsem).wait()

    @pl.when(i + 3 < NCH)
    def _():
      pltpu.make_async_copy(src_hbm.at[pl.ds(base, EC)], sidx[p],
                            isem[p]).wait()
      pltpu.async_copy(x_hbm.at[sidx[p]], rowsb[p], gsem[p])
      pltpu.async_copy(dst_hbm.at[pl.ds(base + (i + 3) * EC, EC)],
                       didx[p], jsem[p])

    @pl.when(i + 4 < NCH)
    def _():
      pltpu.async_copy(src_hbm.at[pl.ds(base + (i + 4) * EC, EC)],
                       sidx[b], isem[b])

  def _quad(g, _):
    for b in range(4):
      _step(4 * g + b, b)
    return _
  lax.fori_loop(0, NCH // 4, _quad, None)
  _step(NCH - 1, 0)

  pltpu.make_async_copy(rowsb[0], acc.at[didx[0]], ssem[0]).wait()
  pltpu.make_async_copy(ones, dacc.at[didx[0]], dsem).wait()

  plsc.subcore_barrier()

  sl = pl.ds(sid * RPT, RPT)
  pltpu.sync_copy(acc.at[sl], psum_hbm.at[cid, sl])
  pltpu.sync_copy(dacc.at[sl], pdeg_hbm.at[pl.ds(cid * NPAD + sid * RPT, RPT)])


@functools.partial(
    pl.kernel,
    out_type=jax.ShapeDtypeStruct((NPAD, D), jnp.float32),
    mesh=plsc.VectorSubcoreMesh(
        core_axis_name="c", subcore_axis_name="s", num_cores=NC,
        num_subcores=NS),
    scratch_types=[
        pltpu.VMEM((CB, D), jnp.float32),
        pltpu.VMEM((CB, D), jnp.float32),
        pltpu.VMEM((CB, D), jnp.float32),
        pltpu.VMEM((CB, D), jnp.float32),
        pltpu.VMEM((RB,), jnp.float32),
        pltpu.VMEM((RB,), jnp.float32),
    ],
)
def _sc_combine(x_hbm, psum_hbm, pdeg_hbm, agg_hbm,
                xb, p0b, p1b, ob, d0b, d1b):
  cid = lax.axis_index("c")
  sid = lax.axis_index("s")
  wid = cid * NS + sid
  base = wid * RB

  pltpu.sync_copy(pdeg_hbm.at[pl.ds(base, RB)], d0b)
  pltpu.sync_copy(pdeg_hbm.at[pl.ds(NPAD + base, RB)], d1b)

  for ch in range(RB // CB):
    rb = base + ch * CB
    pltpu.sync_copy(x_hbm.at[pl.ds(rb, CB)], xb)
    pltpu.sync_copy(psum_hbm.at[0, pl.ds(rb, CB)], p0b)
    pltpu.sync_copy(psum_hbm.at[1, pl.ds(rb, CB)], p1b)

    def _rowgrp(g, _):
      d = d0b[pl.ds(ch * CB + g * 16, 16)] + d1b[pl.ds(ch * CB + g * 16, 16)]
      has = d > 0.0
      sn = jnp.where(has, 0.5 / jnp.maximum(d, 1.0), 0.0)
      sx = jnp.where(has, 0.5, 1.0)
      for j in range(16):
        snj = sn[j]
        sxj = sx[j]
        for c in range(D // 16):
          sl = pl.ds(c * 16, 16)
          r = g * 16 + j
          ob[r, sl] = xb[r, sl] * sxj + (p0b[r, sl] + p1b[r, sl]) * snj
      return _
    lax.fori_loop(0, CB // 16, _rowgrp, None)

    pltpu.sync_copy(ob, agg_hbm.at[pl.ds(rb, CB)])


BN = 2000


def _mlp_body(a_ref, w1_ref, b1_ref, w2_ref, b2_ref, w3_ref, b3_ref, o_ref):
  dn = (((1,), (1,)), ((), ()))
  a = a_ref[...]
  h = lax.dot_general(a, w1_ref[...], dn, preferred_element_type=jnp.float32)
  h = jnp.maximum(h + b1_ref[...], 0.0)
  h = lax.dot_general(h, w2_ref[...], dn, preferred_element_type=jnp.float32)
  h = jnp.maximum(h + b2_ref[...], 0.0)
  o_ref[...] = jnp.sum(h * w3_ref[...], axis=1, keepdims=True) + b3_ref[...]


def _tc_mlp(agg, W1, b1, W2, b2, W3, b3):
  return pl.pallas_call(
      _mlp_body,
      grid=(N // BN,),
      in_specs=[
          pl.BlockSpec((BN, D), lambda g: (g, 0)),
          pl.BlockSpec((H, D), lambda g: (0, 0)),
          pl.BlockSpec((1, H), lambda g: (0, 0)),
          pl.BlockSpec((H, H), lambda g: (0, 0)),
          pl.BlockSpec((1, H), lambda g: (0, 0)),
          pl.BlockSpec((1, H), lambda g: (0, 0)),
          pl.BlockSpec((1, 1), lambda g: (0, 0)),
      ],
      out_specs=pl.BlockSpec((BN, 1), lambda g: (g, 0)),
      out_shape=jax.ShapeDtypeStruct((N, 1), jnp.float32),
  )(agg, W1, b1.reshape(1, H), W2, b2.reshape(1, H), W3, b3.reshape(1, 1))


def kernel(x, edge_index, W1, b1, W2, b2, W3, b3):
  src = edge_index[0]
  dst = edge_index[1]
  x_pad = jnp.pad(x, ((0, NPAD - N), (0, 0)))
  psum, pdeg = _sc_accumulate(x_pad, src, dst)
  agg = _sc_combine(x_pad, psum, pdeg)
  return _tc_mlp(agg[:N], W1, b1, W2, b2, W3, b3)


# speedup vs baseline: 3.1848x; 1.1951x over previous
import functools

import jax
import jax.numpy as jnp
from jax import lax
from jax.experimental import pallas as pl
from jax.experimental.pallas import tpu as pltpu
from jax.experimental.pallas import tpu_sc as plsc

N = 10000
E = 320000
D = 128
H = 128

NC = 2
NS = 16
NW = NC * NS

NPAD = 10240
RPT = NPAD // NS
EW = E // NW
EC = 80
NCH = EW // EC

@functools.partial(
    pl.kernel,
    out_type=[
        jax.ShapeDtypeStruct((NC, NPAD, D), jnp.float32),
        jax.ShapeDtypeStruct((NC, NPAD, D), jnp.float32),
    ],
    mesh=plsc.VectorSubcoreMesh(
        core_axis_name="c", subcore_axis_name="s", num_cores=NC,
        num_subcores=NS),
    scratch_types=[
        [pltpu.VMEM((EC,), jnp.int32)] * 4,
        [pltpu.VMEM((EC,), jnp.int32)] * 4,
        [pltpu.VMEM((EC, D), jnp.float32)] * 4,
        pltpu.VMEM((EC,), jnp.float32),
        pltpu.VMEM((RPT,), jnp.float32),
        pltpu.VMEM_SHARED((NPAD, D), jnp.float32),
        pltpu.VMEM_SHARED((NPAD,), jnp.float32),
        [pltpu.SemaphoreType.DMA] * 4,
        [pltpu.SemaphoreType.DMA] * 4,
        [pltpu.SemaphoreType.DMA] * 4,
        [pltpu.SemaphoreType.DMA] * 4,
        pltpu.SemaphoreType.DMA,
    ],
)
def _sc_accumulate(x_hbm, src_hbm, dst_hbm, psum_hbm, pdeg_hbm,
                   sidx, didx, rowsb, ones, dzero, acc, dacc,
                   gsem, ssem, isem, jsem, dsem):
  cid = lax.axis_index("c")
  sid = lax.axis_index("s")
  wid = cid * NS + sid
  base = wid * EW
  rows0 = rowsb[0]

  def _zrow(r, _):
    for c in range(D // 16):
      rows0[r, pl.ds(c * 16, 16)] = jnp.zeros((16,), jnp.float32)
    return _
  lax.fori_loop(0, EC, _zrow, None)
  for k in range(RPT // EC):
    pltpu.sync_copy(rows0, acc.at[pl.ds(sid * RPT + k * EC, EC)])

  def _zdeg(i, _):
    dzero[pl.ds(i * 16, 16)] = jnp.zeros((16,), jnp.float32)
    return _
  lax.fori_loop(0, RPT // 16, _zdeg, None)
  pltpu.sync_copy(dzero, dacc.at[pl.ds(sid * RPT, RPT)])

  for i in range(EC // 16):
    ones[pl.ds(i * 16, 16)] = jnp.ones((16,), jnp.float32)

  plsc.subcore_barrier()

  for k in range(3):
    pltpu.sync_copy(src_hbm.at[pl.ds(base + k * EC, EC)], sidx[k])
    pltpu.sync_copy(dst_hbm.at[pl.ds(base + k * EC, EC)], didx[k])
    pltpu.async_copy(x_hbm.at[sidx[k]], rowsb[k], gsem[k])
  pltpu.async_copy(src_hbm.at[pl.ds(base + 3 * EC, EC)], sidx[3], isem[3])

  def _step(i, b):
    p = (b + 3) % 4
    pltpu.make_async_copy(x_hbm.at[sidx[b]], rowsb[b], gsem[b]).wait()

    @pl.when(i >= 3)
    def _():
      pltpu.make_async_copy(dst_hbm.at[pl.ds(base, EC)], didx[b],
                            jsem[b]).wait()

    pltpu.async_copy(rowsb[b], acc.at[didx[b]], ssem[b], add=True)
    pltpu.async_copy(ones, dacc.at[didx[b]], dsem, add=True)

    @pl.when(i > 0)
    def _():
      pltpu.make_async_copy(rowsb[p], acc.at[didx[p]], ssem[p]).wait()
      pltpu.make_async_copy(ones, dacc.at[didx[p]], dsem).wait()

    @pl.when(i + 3 < NCH)
    def _():
      pltpu.make_async_copy(src_hbm.at[pl.ds(base, EC)], sidx[p],
                            isem[p]).wait()
      pltpu.async_copy(x_hbm.at[sidx[p]], rowsb[p], gsem[p])
      pltpu.async_copy(dst_hbm.at[pl.ds(base + (i + 3) * EC, EC)],
                       didx[p], jsem[p])

    @pl.when(i + 4 < NCH)
    def _():
      pltpu.async_copy(src_hbm.at[pl.ds(base + (i + 4) * EC, EC)],
                       sidx[b], isem[b])

  def _quad(g, _):
    for b in range(4):
      _step(4 * g + b, b)
    return _
  lax.fori_loop(0, NCH // 4, _quad, None)
  _step(NCH - 1, 0)

  pltpu.make_async_copy(rowsb[0], acc.at[didx[0]], ssem[0]).wait()
  pltpu.make_async_copy(ones, dacc.at[didx[0]], dsem).wait()

  plsc.subcore_barrier()

  sl = pl.ds(sid * RPT, RPT)
  pltpu.sync_copy(acc.at[sl], psum_hbm.at[cid, sl])
  pltpu.sync_copy(dacc.at[sl], dzero)

  for k in range(RPT // EC):
    buf = rowsb[1 + (k % 2)]
    xsem = gsem[1 + (k % 2)]
    if k >= 2:
      pltpu.make_async_copy(
          buf, pdeg_hbm.at[cid, pl.ds(sid * RPT + (k - 2) * EC, EC)],
          xsem).wait()

    def _bg(g, _):
      dv = dzero[pl.ds(k * EC + g * 16, 16)]
      for j in range(16):
        s = jnp.full((16,), dv[j], jnp.float32)
        for c in range(D // 16):
          buf[g * 16 + j, pl.ds(c * 16, 16)] = s
      return _
    lax.fori_loop(0, EC // 16, _bg, None)
    pltpu.async_copy(buf, pdeg_hbm.at[cid, pl.ds(sid * RPT + k * EC, EC)],
                     xsem)

  for k in range(RPT // EC - 2, RPT // EC):
    pltpu.make_async_copy(
        rowsb[1 + (k % 2)],
        pdeg_hbm.at[cid, pl.ds(sid * RPT + k * EC, EC)],
        gsem[1 + (k % 2)]).wait()


BN = 2048


def _mlp_body(x_ref, ps_ref, pd_ref, w1_ref, b1_ref, w2_ref, b2_ref, w3_ref,
              b3_ref, o_ref):
  deg = pd_ref[0] + pd_ref[1]
  has = deg > 0.0
  sn = jnp.where(has, 0.5 / jnp.maximum(deg, 1.0), 0.0)
  sx = jnp.where(has, 0.5, 1.0)
  a = x_ref[...] * sx + (ps_ref[0] + ps_ref[1]) * sn
  dn = (((1,), (1,)), ((), ()))
  h = lax.dot_general(a, w1_ref[...], dn, preferred_element_type=jnp.float32)
  h = jnp.maximum(h + b1_ref[...], 0.0)
  h = lax.dot_general(h, w2_ref[...], dn, preferred_element_type=jnp.float32)
  h = jnp.maximum(h + b2_ref[...], 0.0)
  o_ref[...] = jnp.sum(h * w3_ref[...], axis=1, keepdims=True) + b3_ref[...]


def _tc_mlp(x_pad, psum, pdegx, W1, b1, W2, b2, W3, b3):
  return pl.pallas_call(
      _mlp_body,
      grid=(NPAD // BN,),
      in_specs=[
          pl.BlockSpec((BN, D), lambda g: (g, 0)),
          pl.BlockSpec((NC, BN, D), lambda g: (0, g, 0)),
          pl.BlockSpec((NC, BN, D), lambda g: (0, g, 0)),
          pl.BlockSpec((H, D), lambda g: (0, 0)),
          pl.BlockSpec((1, H), lambda g: (0, 0)),
          pl.BlockSpec((H, H), lambda g: (0, 0)),
          pl.BlockSpec((1, H), lambda g: (0, 0)),
          pl.BlockSpec((1, H), lambda g: (0, 0)),
          pl.BlockSpec((1, 1), lambda g: (0, 0)),
      ],
      out_specs=pl.BlockSpec((BN, 1), lambda g: (g, 0)),
      out_shape=jax.ShapeDtypeStruct((NPAD, 1), jnp.float32),
  )(x_pad, psum, pdegx, W1, b1.reshape(1, H), W2, b2.reshape(1, H), W3,
    b3.reshape(1, 1))


def kernel(x, edge_index, W1, b1, W2, b2, W3, b3):
  src = edge_index[0]
  dst = edge_index[1]
  x_pad = jnp.pad(x, ((0, NPAD - N), (0, 0)))
  psum, pdegx = _sc_accumulate(x_pad, src, dst)
  out = _tc_mlp(x_pad, psum, pdegx, W1, b1, W2, b2, W3, b3)
  return out[:N]


# speedup vs baseline: 3.2806x; 1.0301x over previous
import functools

import jax
import jax.numpy as jnp
from jax import lax
from jax.experimental import pallas as pl
from jax.experimental.pallas import tpu as pltpu
from jax.experimental.pallas import tpu_sc as plsc

N = 10000
E = 320000
D = 128
H = 128

NC = 2
NS = 16
NW = NC * NS

NPAD = 10240
RPT = NPAD // NS
EW = E // NW
EC = 80
NCH = EW // EC

@functools.partial(
    pl.kernel,
    out_type=[
        jax.ShapeDtypeStruct((NC, NPAD, D), jnp.float32),
        jax.ShapeDtypeStruct((NC, NPAD, D), jnp.float32),
    ],
    mesh=plsc.VectorSubcoreMesh(
        core_axis_name="c", subcore_axis_name="s", num_cores=NC,
        num_subcores=NS),
    scratch_types=[
        [pltpu.VMEM((EC,), jnp.int32)] * 4,
        [pltpu.VMEM((EC,), jnp.int32)] * 4,
        [pltpu.VMEM((EC, D), jnp.float32)] * 4,
        pltpu.VMEM((EC,), jnp.float32),
        pltpu.VMEM((RPT,), jnp.float32),
        pltpu.VMEM_SHARED((NPAD, D), jnp.float32),
        pltpu.VMEM_SHARED((NPAD,), jnp.float32),
        [pltpu.SemaphoreType.DMA] * 4,
        [pltpu.SemaphoreType.DMA] * 4,
        [pltpu.SemaphoreType.DMA] * 4,
        [pltpu.SemaphoreType.DMA] * 4,
        pltpu.SemaphoreType.DMA,
    ],
)
def _sc_accumulate(x_hbm, src_hbm, dst_hbm, psum_hbm, pdeg_hbm,
                   sidx, didx, rowsb, ones, dzero, acc, dacc,
                   gsem, ssem, isem, jsem, dsem):
  cid = lax.axis_index("c")
  sid = lax.axis_index("s")
  wid = cid * NS + sid
  base = wid * EW
  rows0 = rowsb[0]

  def _zrow(r, _):
    for c in range(D // 16):
      rows0[r, pl.ds(c * 16, 16)] = jnp.zeros((16,), jnp.float32)
    return _
  lax.fori_loop(0, EC, _zrow, None)
  for k in range(RPT // EC):
    pltpu.sync_copy(rows0, acc.at[pl.ds(sid * RPT + k * EC, EC)])

  def _zdeg(i, _):
    dzero[pl.ds(i * 16, 16)] = jnp.zeros((16,), jnp.float32)
    return _
  lax.fori_loop(0, RPT // 16, _zdeg, None)
  pltpu.sync_copy(dzero, dacc.at[pl.ds(sid * RPT, RPT)])

  for i in range(EC // 16):
    ones[pl.ds(i * 16, 16)] = jnp.ones((16,), jnp.float32)

  plsc.subcore_barrier()

  for k in range(3):
    pltpu.sync_copy(src_hbm.at[pl.ds(base + k * EC, EC)], sidx[k])
    pltpu.sync_copy(dst_hbm.at[pl.ds(base + k * EC, EC)], didx[k])
    pltpu.async_copy(x_hbm.at[sidx[k]], rowsb[k], gsem[k])
  pltpu.async_copy(src_hbm.at[pl.ds(base + 3 * EC, EC)], sidx[3], isem[3])

  def _step(i, b):
    p = (b + 3) % 4
    pltpu.make_async_copy(x_hbm.at[sidx[b]], rowsb[b], gsem[b]).wait()

    @pl.when(i >= 3)
    def _():
      pltpu.make_async_copy(dst_hbm.at[pl.ds(base, EC)], didx[b],
                            jsem[b]).wait()

    pltpu.async_copy(rowsb[b], acc.at[didx[b]], ssem[b], add=True)
    pltpu.async_copy(ones, dacc.at[didx[b]], dsem, add=True)

    @pl.when(i > 0)
    def _():
      pltpu.make_async_copy(rowsb[p], acc.at[didx[p]], ssem[p]).wait()
      pltpu.make_async_copy(ones, dacc.at[didx[p]], dsem).wait()

    @pl.when(i + 3 < NCH)
    def _():
      pltpu.make_async_copy(src_hbm.at[pl.ds(base, EC)], sidx[p],
                            isem[p]).wait()
      pltpu.async_copy(x_hbm.at[sidx[p]], rowsb[p], gsem[p])
      pltpu.async_copy(dst_hbm.at[pl.ds(base + (i + 3) * EC, EC)],
                       didx[p], jsem[p])

    @pl.when(i + 4 < NCH)
    def _():
      pltpu.async_copy(src_hbm.at[pl.ds(base + (i + 4) * EC, EC)],
                       sidx[b], isem[b])

  def _quad(g, _):
    for b in range(4):
      _step(4 * g + b, b)
    return _
  lax.fori_loop(0, NCH // 4, _quad, None)
  _step(NCH - 1, 0)

  pltpu.make_async_copy(rowsb[0], acc.at[didx[0]], ssem[0]).wait()
  pltpu.make_async_copy(ones, dacc.at[didx[0]], dsem).wait()

  plsc.subcore_barrier()

  sl = pl.ds(sid * RPT, RPT)
  pltpu.async_copy(acc.at[sl], psum_hbm.at[cid, sl], gsem[0])
  pltpu.sync_copy(dacc.at[sl], dzero)

  for k in range(RPT // EC):
    buf = rowsb[1 + (k % 2)]
    xsem = gsem[1 + (k % 2)]
    if k >= 2:
      pltpu.make_async_copy(
          buf, pdeg_hbm.at[cid, pl.ds(sid * RPT + (k - 2) * EC, EC)],
          xsem).wait()

    def _bg(g, _):
      dv = dzero[pl.ds(k * EC + g * 16, 16)]
      for j in range(16):
        s = jnp.full((16,), dv[j], jnp.float32)
        for c in range(D // 16):
          buf[g * 16 + j, pl.ds(c * 16, 16)] = s
      return _
    lax.fori_loop(0, EC // 16, _bg, None)
    pltpu.async_copy(buf, pdeg_hbm.at[cid, pl.ds(sid * RPT + k * EC, EC)],
                     xsem)

  for k in range(RPT // EC - 2, RPT // EC):
    pltpu.make_async_copy(
        rowsb[1 + (k % 2)],
        pdeg_hbm.at[cid, pl.ds(sid * RPT + k * EC, EC)],
        gsem[1 + (k % 2)]).wait()
  pltpu.make_async_copy(acc.at[sl], psum_hbm.at[cid, sl], gsem[0]).wait()


BN = 2048


def _mlp_body(x_ref, ps_ref, pd_ref, w1_ref, b1_ref, w2_ref, b2_ref, w3_ref,
              b3_ref, o_ref):
  deg = pd_ref[0] + pd_ref[1]
  has = deg > 0.0
  sn = jnp.where(has, 0.5 / jnp.maximum(deg, 1.0), 0.0)
  sx = jnp.where(has, 0.5, 1.0)
  a = x_ref[...] * sx + (ps_ref[0] + ps_ref[1]) * sn
  dn = (((1,), (1,)), ((), ()))
  bf = jnp.bfloat16
  h = lax.dot_general(a.astype(bf), w1_ref[...].astype(bf), dn,
                      preferred_element_type=jnp.float32)
  h = jnp.maximum(h + b1_ref[...], 0.0)
  h = lax.dot_general(h.astype(bf), w2_ref[...].astype(bf), dn,
                      preferred_element_type=jnp.float32)
  h = jnp.maximum(h + b2_ref[...], 0.0)
  o_ref[...] = jnp.sum(h * w3_ref[...], axis=1, keepdims=True) + b3_ref[...]


def _tc_mlp(x_pad, psum, pdegx, W1, b1, W2, b2, W3, b3):
  return pl.pallas_call(
      _mlp_body,
      grid=(NPAD // BN,),
      in_specs=[
          pl.BlockSpec((BN, D), lambda g: (g, 0)),
          pl.BlockSpec((NC, BN, D), lambda g: (0, g, 0)),
          pl.BlockSpec((NC, BN, D), lambda g: (0, g, 0)),
          pl.BlockSpec((H, D), lambda g: (0, 0)),
          pl.BlockSpec((1, H), lambda g: (0, 0)),
          pl.BlockSpec((H, H), lambda g: (0, 0)),
          pl.BlockSpec((1, H), lambda g: (0, 0)),
          pl.BlockSpec((1, H), lambda g: (0, 0)),
          pl.BlockSpec((1, 1), lambda g: (0, 0)),
      ],
      out_specs=pl.BlockSpec((BN, 1), lambda g: (g, 0)),
      out_shape=jax.ShapeDtypeStruct((NPAD, 1), jnp.float32),
  )(x_pad, psum, pdegx, W1, b1.reshape(1, H), W2, b2.reshape(1, H), W3,
    b3.reshape(1, 1))


def kernel(x, edge_index, W1, b1, W2, b2, W3, b3):
  src = edge_index[0]
  dst = edge_index[1]
  x_pad = jnp.pad(x, ((0, NPAD - N), (0, 0)))
  psum, pdegx = _sc_accumulate(x_pad, src, dst)
  out = _tc_mlp(x_pad, psum, pdegx, W1, b1, W2, b2, W3, b3)
  return out[:N]


# speedup vs baseline: 3.3648x; 1.0257x over previous
import functools

import jax
import jax.numpy as jnp
from jax import lax
from jax.experimental import pallas as pl
from jax.experimental.pallas import tpu as pltpu
from jax.experimental.pallas import tpu_sc as plsc

N = 10000
E = 320000
D = 128
H = 128

NC = 2
NS = 16
NW = NC * NS

NPAD = 10240
RPT = NPAD // NS
EW = E // NW
EC = 80
NCH = EW // EC

@functools.partial(
    pl.kernel,
    out_type=[
        jax.ShapeDtypeStruct((NC, NPAD, D), jnp.float32),
        jax.ShapeDtypeStruct((NC, NPAD, D), jnp.float32),
    ],
    mesh=plsc.VectorSubcoreMesh(
        core_axis_name="c", subcore_axis_name="s", num_cores=NC,
        num_subcores=NS),
    scratch_types=[
        [pltpu.VMEM((EC,), jnp.int32)] * 4,
        [pltpu.VMEM((EC,), jnp.int32)] * 4,
        [pltpu.VMEM((EC, D), jnp.float32)] * 4,
        pltpu.VMEM((EC,), jnp.float32),
        pltpu.VMEM((RPT,), jnp.float32),
        pltpu.VMEM_SHARED((NPAD, D), jnp.float32),
        pltpu.VMEM_SHARED((NPAD,), jnp.float32),
        [pltpu.SemaphoreType.DMA] * 4,
        [pltpu.SemaphoreType.DMA] * 4,
        [pltpu.SemaphoreType.DMA] * 4,
        [pltpu.SemaphoreType.DMA] * 4,
        pltpu.SemaphoreType.DMA,
    ],
)
def _sc_accumulate(x_hbm, src_hbm, dst_hbm, psum_hbm, pdeg_hbm,
                   sidx, didx, rowsb, ones, dzero, acc, dacc,
                   gsem, ssem, isem, jsem, dsem):
  cid = lax.axis_index("c")
  sid = lax.axis_index("s")
  wid = cid * NS + sid
  base = wid * EW
  rows0 = rowsb[0]

  def _zrow(r, _):
    for c in range(D // 16):
      rows0[r, pl.ds(c * 16, 16)] = jnp.zeros((16,), jnp.float32)
    return _
  lax.fori_loop(0, EC, _zrow, None)
  for k in range(RPT // EC):
    pltpu.async_copy(rows0, acc.at[pl.ds(sid * RPT + k * EC, EC)], ssem[0])

  def _zdeg(i, _):
    dzero[pl.ds(i * 16, 16)] = jnp.zeros((16,), jnp.float32)
    return _
  lax.fori_loop(0, RPT // 16, _zdeg, None)
  pltpu.async_copy(dzero, dacc.at[pl.ds(sid * RPT, RPT)], ssem[1])

  for i in range(EC // 16):
    ones[pl.ds(i * 16, 16)] = jnp.ones((16,), jnp.float32)

  for k in range(RPT // EC):
    pltpu.make_async_copy(rows0, acc.at[pl.ds(sid * RPT, EC)], ssem[0]).wait()
  pltpu.make_async_copy(dzero, dacc.at[pl.ds(sid * RPT, RPT)], ssem[1]).wait()

  plsc.subcore_barrier()

  for k in range(3):
    pltpu.sync_copy(src_hbm.at[pl.ds(base + k * EC, EC)], sidx[k])
    pltpu.sync_copy(dst_hbm.at[pl.ds(base + k * EC, EC)], didx[k])
    pltpu.async_copy(x_hbm.at[sidx[k]], rowsb[k], gsem[k])
  pltpu.async_copy(src_hbm.at[pl.ds(base + 3 * EC, EC)], sidx[3], isem[3])

  def _step(i, b):
    p = (b + 3) % 4
    pltpu.make_async_copy(x_hbm.at[sidx[b]], rowsb[b], gsem[b]).wait()

    @pl.when(i >= 3)
    def _():
      pltpu.make_async_copy(dst_hbm.at[pl.ds(base, EC)], didx[b],
                            jsem[b]).wait()

    pltpu.async_copy(rowsb[b], acc.at[didx[b]], ssem[b], add=True)
    pltpu.async_copy(ones, dacc.at[didx[b]], dsem, add=True)

    @pl.when(i > 0)
    def _():
      pltpu.make_async_copy(rowsb[p], acc.at[didx[p]], ssem[p]).wait()
      pltpu.make_async_copy(ones, dacc.at[didx[p]], dsem).wait()

    @pl.when(i + 3 < NCH)
    def _():
      pltpu.make_async_copy(src_hbm.at[pl.ds(base, EC)], sidx[p],
                            isem[p]).wait()
      pltpu.async_copy(x_hbm.at[sidx[p]], rowsb[p], gsem[p])
      pltpu.async_copy(dst_hbm.at[pl.ds(base + (i + 3) * EC, EC)],
                       didx[p], jsem[p])

    @pl.when(i + 4 < NCH)
    def _():
      pltpu.async_copy(src_hbm.at[pl.ds(base + (i + 4) * EC, EC)],
                       sidx[b], isem[b])

  def _quad(g, _):
    for b in range(4):
      _step(4 * g + b, b)
    return _
  lax.fori_loop(0, NCH // 4, _quad, None)
  _step(NCH - 1, 0)

  pltpu.make_async_copy(rowsb[0], acc.at[didx[0]], ssem[0]).wait()
  pltpu.make_async_copy(ones, dacc.at[didx[0]], dsem).wait()

  plsc.subcore_barrier()

  sl = pl.ds(sid * RPT, RPT)
  pltpu.async_copy(acc.at[sl], psum_hbm.at[cid, sl], gsem[0])
  pltpu.sync_copy(dacc.at[sl], dzero)

  for k in range(RPT // EC):
    buf = rowsb[1 + (k % 2)]
    xsem = gsem[1 + (k % 2)]
    if k >= 2:
      pltpu.make_async_copy(
          buf, pdeg_hbm.at[cid, pl.ds(sid * RPT + (k - 2) * EC, EC)],
          xsem).wait()

    def _bg(g, _):
      dv = dzero[pl.ds(k * EC + g * 16, 16)]
      for j in range(16):
        s = jnp.full((16,), dv[j], jnp.float32)
        for c in range(D // 16):
          buf[g * 16 + j, pl.ds(c * 16, 16)] = s
      return _
    lax.fori_loop(0, EC // 16, _bg, None)
    pltpu.async_copy(buf, pdeg_hbm.at[cid, pl.ds(sid * RPT + k * EC, EC)],
                     xsem)

  for k in range(RPT // EC - 2, RPT // EC):
    pltpu.make_async_copy(
        rowsb[1 + (k % 2)],
        pdeg_hbm.at[cid, pl.ds(sid * RPT + k * EC, EC)],
        gsem[1 + (k % 2)]).wait()
  pltpu.make_async_copy(acc.at[sl], psum_hbm.at[cid, sl], gsem[0]).wait()


BN = 2000


def _mlp_body(x_ref, ps_ref, pd_ref, w1_ref, b1_ref, w2_ref, b2_ref, w3_ref,
              b3_ref, o_ref):
  deg = pd_ref[0] + pd_ref[1]
  has = deg > 0.0
  sn = jnp.where(has, 0.5 / jnp.maximum(deg, 1.0), 0.0)
  sx = jnp.where(has, 0.5, 1.0)
  a = x_ref[...] * sx + (ps_ref[0] + ps_ref[1]) * sn
  dn = (((1,), (1,)), ((), ()))
  bf = jnp.bfloat16
  h = lax.dot_general(a.astype(bf), w1_ref[...].astype(bf), dn,
                      preferred_element_type=jnp.float32)
  h = jnp.maximum(h + b1_ref[...], 0.0)
  h = lax.dot_general(h.astype(bf), w2_ref[...].astype(bf), dn,
                      preferred_element_type=jnp.float32)
  h = jnp.maximum(h + b2_ref[...], 0.0)
  o_ref[...] = jnp.sum(h * w3_ref[...], axis=1, keepdims=True) + b3_ref[...]


def _tc_mlp(x, psum, pdegx, W1, b1, W2, b2, W3, b3):
  return pl.pallas_call(
      _mlp_body,
      grid=(N // BN,),
      in_specs=[
          pl.BlockSpec((BN, D), lambda g: (g, 0)),
          pl.BlockSpec((NC, BN, D), lambda g: (0, g, 0)),
          pl.BlockSpec((NC, BN, D), lambda g: (0, g, 0)),
          pl.BlockSpec((H, D), lambda g: (0, 0)),
          pl.BlockSpec((1, H), lambda g: (0, 0)),
          pl.BlockSpec((H, H), lambda g: (0, 0)),
          pl.BlockSpec((1, H), lambda g: (0, 0)),
          pl.BlockSpec((1, H), lambda g: (0, 0)),
          pl.BlockSpec((1, 1), lambda g: (0, 0)),
      ],
      out_specs=pl.BlockSpec((BN, 1), lambda g: (g, 0)),
      out_shape=jax.ShapeDtypeStruct((N, 1), jnp.float32),
  )(x, psum, pdegx, W1, b1.reshape(1, H), W2, b2.reshape(1, H), W3,
    b3.reshape(1, 1))


def kernel(x, edge_index, W1, b1, W2, b2, W3, b3):
  src = edge_index[0]
  dst = edge_index[1]
  psum, pdegx = _sc_accumulate(x, src, dst)
  return _tc_mlp(x, psum, pdegx, W1, b1, W2, b2, W3, b3)
